# baseline (speedup 1.0000x reference)
import functools

import jax
import jax.numpy as jnp
from jax import lax
from jax.experimental import pallas as pl
from jax.experimental.pallas import tpu as pltpu
from jax.experimental.pallas import tpu_sc as plsc

BATCH = 16384
NF = 26
VOCAB = 100000
D = 32

NC = 2
NS = 16
L = 16
NW = NC * NS

R = BATCH * NF
RW = R // NW
JROWS = RW // 128
CROWS = 1664
NGRP = RW // CROWS
KJ = CROWS // 128

VB = 512
WB = VB // 4
NVB = -(-VOCAB // VB)
FW = NVB * WB


def _transpose_body(in_ref, out_ref):
    t = in_ref[0]
    tt = t.T
    for s in range(4):
        out_ref[:, pl.ds(s * D, D)] = tt[s * WB:(s + 1) * WB, :]


def _emb_body(x_hbm, tab_hbm, out_hbm, xidx, rows, semA, semB, semSA, semSB):
    wid = lax.axis_index("s") * NC + lax.axis_index("c")

    pltpu.sync_copy(x_hbm.at[pl.ds(wid * JROWS, JROWS), :], xidx)

    def cvt(j, carry):
        for o in range(128 // L):
            pos = j * 128 + o * L + lax.iota(jnp.int32, L)
            fid = pos % NF
            vloc = xidx[j, pl.ds(o * L, L)]
            q = vloc >> 9
            r = vloc & (VB - 1)
            w = r & (WB - 1)
            s = r >> 7
            xidx[j, pl.ds(o * L, L)] = fid * (FW * 4) + (q << 9) + (w << 2) + s
        return carry

    lax.fori_loop(0, JROWS, cvt, 0)

    gsem = [semA, semB]
    ssem = [semSA, semSB]

    def fire(g):
        buf, sem = g % 2, gsem[g % 2]
        return [
            pltpu.async_copy(tab_hbm.at[xidx.at[g * KJ + k]],
                             rows.at[buf, pl.ds(k * 128, 128), :], sem)
            for k in range(KJ)
        ]

    def fire_store(g):
        buf = g % 2
        return pltpu.async_copy(
            rows.at[buf],
            out_hbm.at[pl.ds(wid * RW + g * CROWS, CROWS), :], ssem[buf])

    gathers = {0: fire(0)}
    stores = {}
    for g in range(1, NGRP):
        if g >= 2:
            stores[g - 2].wait()
        gathers[g] = fire(g)
        for h in gathers[g - 1]:
            h.wait()
        stores[g - 1] = fire_store(g - 1)
    for h in gathers[NGRP - 1]:
        h.wait()
    stores[NGRP - 1] = fire_store(NGRP - 1)
    stores[NGRP - 2].wait()
    stores[NGRP - 1].wait()


@jax.jit
def kernel(x, tables):
    x2d = x.reshape(R // 128, 128)
    y = tables.transpose(0, 2, 1)
    lin = pl.pallas_call(
        _transpose_body,
        grid=(NF, NVB),
        in_specs=[pl.BlockSpec((1, D, VB), lambda f, vb: (f, 0, vb))],
        out_specs=pl.BlockSpec((WB, 128), lambda f, vb: (f * NVB + vb, 0)),
        out_shape=jax.ShapeDtypeStruct((NF * FW, 128), jnp.float32),
    )(y)
    tab = lin.reshape(NF * FW * 4, D)
    mesh = plsc.VectorSubcoreMesh(core_axis_name="c", subcore_axis_name="s")
    out = pl.kernel(
        _emb_body,
        out_type=jax.ShapeDtypeStruct((R, D), jnp.float32),
        mesh=mesh,
        scratch_types=[
            pltpu.VMEM((JROWS, 128), jnp.int32),
            pltpu.VMEM((2, CROWS, D), jnp.float32),
            pltpu.SemaphoreType.DMA,
            pltpu.SemaphoreType.DMA,
            pltpu.SemaphoreType.DMA,
            pltpu.SemaphoreType.DMA,
        ],
        compiler_params=pltpu.CompilerParams(use_tc_tiling_on_sc=False),
    )(x2d, tab)
    return out.reshape(BATCH, NF * D)


# speedup vs baseline: 2.9793x; 2.9793x over previous
import functools

import jax
import jax.numpy as jnp
from jax import lax
from jax.experimental import pallas as pl
from jax.experimental.pallas import tpu as pltpu
from jax.experimental.pallas import tpu_sc as plsc

BATCH = 16384
NF = 26
VOCAB = 100000
D = 32

NC = 2
NS = 16
L = 16
NW = NC * NS

R = BATCH * NF
RW = R // NW
JROWS = RW // 128
CROWS = 1664
NGRP = RW // CROWS
KJ = CROWS // 128

VB = 512
NVB = -(-VOCAB // VB)
FR = 7
TW = FR * NVB * VB


def _transpose_body(in_ref, out_ref):
    out_ref[...] = in_ref[...].T


def _emb_body(x_hbm, tab_hbm, out_hbm, xidx, rows, semA, semB, semSA, semSB):
    wid = lax.axis_index("s") * NC + lax.axis_index("c")

    pltpu.sync_copy(x_hbm.at[pl.ds(wid * JROWS, JROWS), :], xidx)

    def cvt(j, carry):
        for o in range(128 // L):
            pos = j * 128 + o * L + lax.iota(jnp.int32, L)
            fid = pos % NF
            vloc = xidx[j, pl.ds(o * L, L)]
            q = vloc >> 9
            vl = vloc & (VB - 1)
            xidx[j, pl.ds(o * L, L)] = (
                (((fid >> 2) * NVB + q) << 11) + (vl << 2) + (fid & 3))
        return carry

    lax.fori_loop(0, JROWS, cvt, 0)

    gsem = [semA, semB]
    ssem = [semSA, semSB]

    def fire(g):
        buf, sem = g % 2, gsem[g % 2]
        return [
            pltpu.async_copy(tab_hbm.at[xidx.at[g * KJ + k]],
                             rows.at[buf, pl.ds(k * 128, 128), :], sem)
            for k in range(KJ)
        ]

    def fire_store(g):
        buf = g % 2
        return pltpu.async_copy(
            rows.at[buf],
            out_hbm.at[pl.ds(wid * RW + g * CROWS, CROWS), :], ssem[buf])

    gathers = {0: fire(0)}
    stores = {}
    for g in range(1, NGRP):
        if g >= 2:
            stores[g - 2].wait()
        gathers[g] = fire(g)
        for h in gathers[g - 1]:
            h.wait()
        stores[g - 1] = fire_store(g - 1)
    for h in gathers[NGRP - 1]:
        h.wait()
    stores[NGRP - 1] = fire_store(NGRP - 1)
    stores[NGRP - 2].wait()
    stores[NGRP - 1].wait()


@jax.jit
def kernel(x, tables):
    x2d = x.reshape(R // 128, 128)
    y = tables.transpose(0, 2, 1).reshape(NF * D, VOCAB)
    lin = pl.pallas_call(
        _transpose_body,
        grid=(FR, NVB),
        in_specs=[pl.BlockSpec((128, VB), lambda fr, vb: (fr, vb))],
        out_specs=pl.BlockSpec((VB, 128), lambda fr, vb: (fr * NVB + vb, 0)),
        out_shape=jax.ShapeDtypeStruct((TW, 128), jnp.float32),
    )(y)
    tab = lin.reshape(TW * 4, D)
    mesh = plsc.VectorSubcoreMesh(core_axis_name="c", subcore_axis_name="s")
    out = pl.kernel(
        _emb_body,
        out_type=jax.ShapeDtypeStruct((R, D), jnp.float32),
        mesh=mesh,
        scratch_types=[
            pltpu.VMEM((JROWS, 128), jnp.int32),
            pltpu.VMEM((2, CROWS, D), jnp.float32),
            pltpu.SemaphoreType.DMA,
            pltpu.SemaphoreType.DMA,
            pltpu.SemaphoreType.DMA,
            pltpu.SemaphoreType.DMA,
        ],
        compiler_params=pltpu.CompilerParams(use_tc_tiling_on_sc=False),
    )(x2d, tab)
    return out.reshape(BATCH, NF * D)


# speedup vs baseline: 5.7941x; 1.9448x over previous
import functools

import jax
import jax.numpy as jnp
from jax import lax
from jax.experimental import pallas as pl
from jax.experimental.pallas import tpu as pltpu
from jax.experimental.pallas import tpu_sc as plsc

BATCH = 16384
NF = 26
VOCAB = 100000
D = 32

NC = 2
NS = 16
L = 16
NW = NC * NS

R = BATCH * NF
RW = R // NW
JROWS = RW // 128
CROWS = 1664
NGRP = RW // CROWS
KJ = CROWS // 128

VB = 2048
NVB = -(-VOCAB // VB)
FR = 7
TW = FR * NVB * VB


def _transpose_body(in_ref, out_ref):
    out_ref[...] = in_ref[...].T


def _emb_body(x_hbm, tab_hbm, out_hbm, xidx, rows, semA, semB, semSA, semSB):
    wid = lax.axis_index("s") * NC + lax.axis_index("c")

    pltpu.sync_copy(x_hbm.at[pl.ds(wid * JROWS, JROWS), :], xidx)

    def cvt(j, carry):
        for o in range(128 // L):
            pos = j * 128 + o * L + lax.iota(jnp.int32, L)
            fid = pos % NF
            vloc = xidx[j, pl.ds(o * L, L)]
            q = vloc >> 11
            vl = vloc & (VB - 1)
            xidx[j, pl.ds(o * L, L)] = (
                (((fid >> 2) * NVB + q) << 13) + (vl << 2) + (fid & 3))
        return carry

    lax.fori_loop(0, JROWS, cvt, 0)

    gsem = [semA, semB]
    ssem = [semSA, semSB]

    def fire(g):
        buf, sem = g % 2, gsem[g % 2]
        return [
            pltpu.async_copy(tab_hbm.at[xidx.at[g * KJ + k]],
                             rows.at[buf, pl.ds(k * 128, 128), :], sem)
            for k in range(KJ)
        ]

    def fire_store(g):
        buf = g % 2
        return pltpu.async_copy(
            rows.at[buf],
            out_hbm.at[pl.ds(wid * RW + g * CROWS, CROWS), :], ssem[buf])

    gathers = {0: fire(0)}
    stores = {}
    for g in range(1, NGRP):
        if g >= 2:
            stores[g - 2].wait()
        gathers[g] = fire(g)
        for h in gathers[g - 1]:
            h.wait()
        stores[g - 1] = fire_store(g - 1)
    for h in gathers[NGRP - 1]:
        h.wait()
    stores[NGRP - 1] = fire_store(NGRP - 1)
    stores[NGRP - 2].wait()
    stores[NGRP - 1].wait()


@jax.jit
def kernel(x, tables):
    x2d = x.reshape(R // 128, 128)
    y = tables.transpose(0, 2, 1).reshape(NF * D, VOCAB)
    lin = pl.pallas_call(
        _transpose_body,
        grid=(FR, NVB),
        in_specs=[pl.BlockSpec((128, VB), lambda fr, vb: (fr, vb))],
        out_specs=pl.BlockSpec((VB, 128), lambda fr, vb: (fr * NVB + vb, 0)),
        out_shape=jax.ShapeDtypeStruct((TW, 128), jnp.float32),
    )(y)
    tab = lin.reshape(TW * 4, D)
    mesh = plsc.VectorSubcoreMesh(core_axis_name="c", subcore_axis_name="s")
    out = pl.kernel(
        _emb_body,
        out_type=jax.ShapeDtypeStruct((R, D), jnp.float32),
        mesh=mesh,
        scratch_types=[
            pltpu.VMEM((JROWS, 128), jnp.int32),
            pltpu.VMEM((2, CROWS, D), jnp.float32),
            pltpu.SemaphoreType.DMA,
            pltpu.SemaphoreType.DMA,
            pltpu.SemaphoreType.DMA,
            pltpu.SemaphoreType.DMA,
        ],
        compiler_params=pltpu.CompilerParams(use_tc_tiling_on_sc=False),
    )(x2d, tab)
    return out.reshape(BATCH, NF * D)


# speedup vs baseline: 7.0477x; 1.2164x over previous
import functools

import jax
import jax.numpy as jnp
from jax import lax
from jax.experimental import pallas as pl
from jax.experimental.pallas import tpu as pltpu
from jax.experimental.pallas import tpu_sc as plsc

BATCH = 16384
NF = 26
VOCAB = 100000
D = 32

NC = 2
NS = 16
L = 16
NW = NC * NS

R = BATCH * NF
RW = R // NW
JROWS = RW // 128
CROWS = 1664
NGRP = RW // CROWS
KJ = CROWS // 128

VB = 4096
NVB = -(-VOCAB // VB)
FR = 7
TW = FR * NVB * VB


def _transpose_body(in_ref, out_ref):
    out_ref[...] = in_ref[...].T


def _emb_body(x_hbm, tab_hbm, out_hbm, xidx, rows, semA, semB, semSA, semSB):
    wid = lax.axis_index("s") * NC + lax.axis_index("c")

    pltpu.sync_copy(x_hbm.at[pl.ds(wid * JROWS, JROWS), :], xidx)

    def cvt(j, carry):
        for o in range(128 // L):
            pos = j * 128 + o * L + lax.iota(jnp.int32, L)
            fid = pos % NF
            vloc = xidx[j, pl.ds(o * L, L)]
            q = vloc >> 12
            vl = vloc & (VB - 1)
            xidx[j, pl.ds(o * L, L)] = (
                (((fid >> 2) * NVB + q) << 14) + (vl << 2) + (fid & 3))
        return carry

    lax.fori_loop(0, JROWS, cvt, 0)

    gsem = [semA, semB]
    ssem = [semSA, semSB]

    def fire(g):
        buf, sem = g % 2, gsem[g % 2]
        return [
            pltpu.async_copy(tab_hbm.at[xidx.at[g * KJ + k]],
                             rows.at[buf, pl.ds(k * 128, 128), :], sem)
            for k in range(KJ)
        ]

    def fire_store(g):
        buf = g % 2
        return pltpu.async_copy(
            rows.at[buf],
            out_hbm.at[pl.ds(wid * RW + g * CROWS, CROWS), :], ssem[buf])

    gathers = {0: fire(0)}
    stores = {}
    for g in range(1, NGRP):
        if g >= 2:
            stores[g - 2].wait()
        gathers[g] = fire(g)
        for h in gathers[g - 1]:
            h.wait()
        stores[g - 1] = fire_store(g - 1)
    for h in gathers[NGRP - 1]:
        h.wait()
    stores[NGRP - 1] = fire_store(NGRP - 1)
    stores[NGRP - 2].wait()
    stores[NGRP - 1].wait()


@jax.jit
def kernel(x, tables):
    x2d = x.reshape(R // 128, 128)
    y = tables.transpose(0, 2, 1).reshape(NF * D, VOCAB)
    lin = pl.pallas_call(
        _transpose_body,
        grid=(FR, NVB),
        in_specs=[pl.BlockSpec((128, VB), lambda fr, vb: (fr, vb))],
        out_specs=pl.BlockSpec((VB, 128), lambda fr, vb: (fr * NVB + vb, 0)),
        out_shape=jax.ShapeDtypeStruct((TW, 128), jnp.float32),
    )(y)
    tab = lin.reshape(TW * 4, D)
    mesh = plsc.VectorSubcoreMesh(core_axis_name="c", subcore_axis_name="s")
    out = pl.kernel(
        _emb_body,
        out_type=jax.ShapeDtypeStruct((R, D), jnp.float32),
        mesh=mesh,
        scratch_types=[
            pltpu.VMEM((JROWS, 128), jnp.int32),
            pltpu.VMEM((2, CROWS, D), jnp.float32),
            pltpu.SemaphoreType.DMA,
            pltpu.SemaphoreType.DMA,
            pltpu.SemaphoreType.DMA,
            pltpu.SemaphoreType.DMA,
        ],
        compiler_params=pltpu.CompilerParams(use_tc_tiling_on_sc=False),
    )(x2d, tab)
    return out.reshape(BATCH, NF * D)


# speedup vs baseline: 7.6616x; 1.0871x over previous
import functools

import jax
import jax.numpy as jnp
from jax import lax
from jax.experimental import pallas as pl
from jax.experimental.pallas import tpu as pltpu
from jax.experimental.pallas import tpu_sc as plsc

BATCH = 16384
NF = 26
VOCAB = 100000
D = 32

NC = 2
NS = 16
L = 16
NW = NC * NS

R = BATCH * NF
RW = R // NW
JROWS = RW // 128
CROWS = 1664
NGRP = RW // CROWS
KJ = CROWS // 128

VB = 8192
NVB = -(-VOCAB // VB)
FR = 7
TW = FR * NVB * VB


def _transpose_body(in_ref, out_ref):
    out_ref[...] = in_ref[...].T


def _emb_body(x_hbm, tab_hbm, out_hbm, xidx, rows, semA, semB, semSA, semSB):
    wid = lax.axis_index("s") * NC + lax.axis_index("c")

    pltpu.sync_copy(x_hbm.at[pl.ds(wid * JROWS, JROWS), :], xidx)

    def cvt(j, carry):
        for o in range(128 // L):
            pos = j * 128 + o * L + lax.iota(jnp.int32, L)
            fid = pos % NF
            vloc = xidx[j, pl.ds(o * L, L)]
            q = vloc >> 13
            vl = vloc & (VB - 1)
            xidx[j, pl.ds(o * L, L)] = (
                (((fid >> 2) * NVB + q) << 15) + (vl << 2) + (fid & 3))
        return carry

    lax.fori_loop(0, JROWS, cvt, 0)

    gsem = [semA, semB]
    ssem = [semSA, semSB]

    def fire(g):
        buf, sem = g % 2, gsem[g % 2]
        return [
            pltpu.async_copy(tab_hbm.at[xidx.at[g * KJ + k]],
                             rows.at[buf, pl.ds(k * 128, 128), :], sem)
            for k in range(KJ)
        ]

    def fire_store(g):
        buf = g % 2
        return pltpu.async_copy(
            rows.at[buf],
            out_hbm.at[pl.ds(wid * RW + g * CROWS, CROWS), :], ssem[buf])

    gathers = {0: fire(0)}
    stores = {}
    for g in range(1, NGRP):
        if g >= 2:
            stores[g - 2].wait()
        gathers[g] = fire(g)
        for h in gathers[g - 1]:
            h.wait()
        stores[g - 1] = fire_store(g - 1)
    for h in gathers[NGRP - 1]:
        h.wait()
    stores[NGRP - 1] = fire_store(NGRP - 1)
    stores[NGRP - 2].wait()
    stores[NGRP - 1].wait()


@jax.jit
def kernel(x, tables):
    x2d = x.reshape(R // 128, 128)
    y = tables.transpose(0, 2, 1).reshape(NF * D, VOCAB)
    lin = pl.pallas_call(
        _transpose_body,
        grid=(FR, NVB),
        in_specs=[pl.BlockSpec((128, VB), lambda fr, vb: (fr, vb))],
        out_specs=pl.BlockSpec((VB, 128), lambda fr, vb: (fr * NVB + vb, 0)),
        out_shape=jax.ShapeDtypeStruct((TW, 128), jnp.float32),
    )(y)
    tab = lin.reshape(TW * 4, D)
    mesh = plsc.VectorSubcoreMesh(core_axis_name="c", subcore_axis_name="s")
    out = pl.kernel(
        _emb_body,
        out_type=jax.ShapeDtypeStruct((R, D), jnp.float32),
        mesh=mesh,
        scratch_types=[
            pltpu.VMEM((JROWS, 128), jnp.int32),
            pltpu.VMEM((2, CROWS, D), jnp.float32),
            pltpu.SemaphoreType.DMA,
            pltpu.SemaphoreType.DMA,
            pltpu.SemaphoreType.DMA,
            pltpu.SemaphoreType.DMA,
        ],
        compiler_params=pltpu.CompilerParams(use_tc_tiling_on_sc=False),
    )(x2d, tab)
    return out.reshape(BATCH, NF * D)


# speedup vs baseline: 7.6658x; 1.0005x over previous
import jax
import jax.numpy as jnp
from jax import lax
from jax.experimental import pallas as pl
from jax.experimental.pallas import tpu as pltpu
from jax.experimental.pallas import tpu_sc as plsc

BATCH = 16384
NF = 26
VOCAB = 100000
D = 32

NC = 2
NS = 16
L = 16
NW = NC * NS

R = BATCH * NF
RW = R // NW
JROWS = RW // 128
CROWS = 1664
NGRP = RW // CROWS
KJ = CROWS // 128

VB = 8192
NVB = -(-VOCAB // VB)
FR = 7
TW = FR * NVB * VB


def _transpose_body(in_ref, out_ref):
    out_ref[...] = in_ref[...].T


def _emb_body(x_hbm, tab_hbm, out_hbm, xidx, rows, semA, semB, semSA, semSB):
    wid = lax.axis_index("s") * NC + lax.axis_index("c")

    pltpu.sync_copy(x_hbm.at[pl.ds(wid * JROWS, JROWS), :], xidx)

    def cvt(j, carry):
        for o in range(128 // L):
            pos = j * 128 + o * L + lax.iota(jnp.int32, L)
            fid = pos % NF
            vloc = xidx[j, pl.ds(o * L, L)]
            q = vloc >> 13
            vl = vloc & (VB - 1)
            xidx[j, pl.ds(o * L, L)] = (
                (((fid >> 2) * NVB + q) << 15) + (vl << 2) + (fid & 3))
        return carry

    lax.fori_loop(0, JROWS, cvt, 0)

    gsem = [semA, semB]
    ssem = [semSA, semSB]

    def fire(g):
        buf, sem = g % 2, gsem[g % 2]
        return [
            pltpu.async_copy(tab_hbm.at[xidx.at[g * KJ + k]],
                             rows.at[buf, pl.ds(k * 128, 128), :], sem)
            for k in range(KJ)
        ]

    def fire_store(g):
        buf = g % 2
        return pltpu.async_copy(
            rows.at[buf],
            out_hbm.at[pl.ds(wid * RW + g * CROWS, CROWS), :], ssem[buf])

    gathers = {0: fire(0)}
    stores = {}
    for g in range(1, NGRP):
        if g >= 2:
            stores[g - 2].wait()
        gathers[g] = fire(g)
        for h in gathers[g - 1]:
            h.wait()
        stores[g - 1] = fire_store(g - 1)
    for h in gathers[NGRP - 1]:
        h.wait()
    stores[NGRP - 1] = fire_store(NGRP - 1)
    stores[NGRP - 2].wait()
    stores[NGRP - 1].wait()


@jax.jit
def kernel(x, tables):
    x2d = x.reshape(R // 128, 128)
    y = tables.transpose(0, 2, 1).reshape(NF * D, VOCAB)
    lin = pl.pallas_call(
        _transpose_body,
        grid=(FR, NVB),
        in_specs=[pl.BlockSpec((128, VB), lambda fr, vb: (fr, vb))],
        out_specs=pl.BlockSpec((VB, 128), lambda fr, vb: (fr * NVB + vb, 0)),
        out_shape=jax.ShapeDtypeStruct((TW, 128), jnp.float32),
    )(y)
    tab = lin.reshape(TW * 4, D)
    mesh = plsc.VectorSubcoreMesh(core_axis_name="c", subcore_axis_name="s")
    out = pl.kernel(
        _emb_body,
        out_type=jax.ShapeDtypeStruct((R, D), jnp.float32),
        mesh=mesh,
        scratch_types=[
            pltpu.VMEM((JROWS, 128), jnp.int32),
            pltpu.VMEM((2, CROWS, D), jnp.float32),
            pltpu.SemaphoreType.DMA,
            pltpu.SemaphoreType.DMA,
            pltpu.SemaphoreType.DMA,
            pltpu.SemaphoreType.DMA,
        ],
        compiler_params=pltpu.CompilerParams(use_tc_tiling_on_sc=False),
    )(x2d, tab)
    return out.reshape(BATCH, NF * D)
